# Initial kernel scaffold; baseline (speedup 1.0000x reference)
import jax
import jax.numpy as jnp
from jax.experimental import pallas as pl


def kernel(x, table):
    raise NotImplementedError("write your pallas kernel here")


# speedup vs baseline: 1.2136x; 1.2136x over previous
import functools

import jax
import jax.numpy as jnp
from jax import lax
from jax.experimental import pallas as pl
from jax.experimental.pallas import tpu as pltpu
from jax.experimental.pallas import tpu_sc as plsc

_info = plsc.get_sparse_core_info()
_NC, _NS = _info.num_cores, _info.num_subcores
_NW = _NC * _NS


def _make_gather(V, D, N, chunk):
    assert N % _NW == 0
    per_w = N // _NW
    assert per_w % chunk == 0
    n_chunks = per_w // chunk
    mesh = plsc.VectorSubcoreMesh(core_axis_name="c", subcore_axis_name="s")

    @functools.partial(
        pl.kernel,
        mesh=mesh,
        out_type=jax.ShapeDtypeStruct((N, D), jnp.float32),
        scratch_types=[
            pltpu.VMEM((chunk,), jnp.int32),
            pltpu.VMEM((chunk, D), jnp.float32),
            pltpu.SemaphoreType.DMA,
        ],
        compiler_params=pltpu.CompilerParams(use_tc_tiling_on_sc=False),
    )
    def gather_kernel(table_hbm, idx_hbm, out_hbm, idx_v, rows_v, sem):
        wid = lax.axis_index("s") * _NC + lax.axis_index("c")
        base = wid * per_w

        def body(i, carry):
            off = base + i * chunk
            pltpu.sync_copy(idx_hbm.at[pl.ds(off, chunk)], idx_v)
            pltpu.async_copy(table_hbm.at[idx_v], rows_v, sem).wait()
            pltpu.sync_copy(rows_v, out_hbm.at[pl.ds(off, chunk)])
            return carry

        lax.fori_loop(0, n_chunks, body, 0)

    return gather_kernel


def _seq_lens_body(x_ref, o_ref):
    o_ref[...] = jnp.sum((x_ref[...] != 0).astype(jnp.int32), axis=1)


def kernel(x, table):
    B_, L_ = x.shape
    V, D = table.shape
    N = B_ * L_
    x_i32 = x.astype(jnp.int32)
    idx = x_i32.reshape(N)

    emb_flat = _make_gather(V, D, N, chunk=128)(table, idx)
    emb = emb_flat.reshape(B_, L_, D)

    seq_lens = pl.pallas_call(
        _seq_lens_body,
        out_shape=jax.ShapeDtypeStruct((B_,), jnp.int32),
    )(x_i32)

    return (emb, seq_lens)


# speedup vs baseline: 1.4764x; 1.2165x over previous
import functools

import jax
import jax.numpy as jnp
from jax import lax
from jax.experimental import pallas as pl
from jax.experimental.pallas import tpu as pltpu
from jax.experimental.pallas import tpu_sc as plsc

_info = plsc.get_sparse_core_info()
_NC, _NS = _info.num_cores, _info.num_subcores
_NW = _NC * _NS

_CHUNK = 128
_K = 5


def _make_gather(V, D, N):
    assert N % (_NW * _CHUNK) == 0
    per_w = N // _NW
    n_chunks = per_w // _CHUNK
    assert n_chunks % (2 * _K) == 0
    n_groups = n_chunks // _K
    mesh = plsc.VectorSubcoreMesh(core_axis_name="c", subcore_axis_name="s")

    scratch = [pltpu.VMEM((n_chunks, _CHUNK), jnp.int32)]
    scratch += [pltpu.VMEM((_CHUNK, D), jnp.float32) for _ in range(2 * _K)]
    scratch += [pltpu.SemaphoreType.DMA for _ in range(4)]

    @functools.partial(
        pl.kernel,
        mesh=mesh,
        out_type=jax.ShapeDtypeStruct((N, D), jnp.float32),
        scratch_types=scratch,
        compiler_params=pltpu.CompilerParams(use_tc_tiling_on_sc=False),
    )
    def gather_kernel(table_hbm, idx_hbm, out_hbm, idx_v, *rest):
        rows = rest[: 2 * _K]
        gsems = rest[2 * _K : 2 * _K + 2]
        wsems = rest[2 * _K + 2 :]
        wid = lax.axis_index("s") * _NC + lax.axis_index("c")
        base = wid * per_w

        pltpu.sync_copy(idx_hbm.at[wid], idx_v)

        def start_gathers(g, h):
            for b in range(_K):
                j = g * _K + b
                pltpu.async_copy(table_hbm.at[idx_v.at[j]], rows[h * _K + b], gsems[h])

        def drain_gathers(h):
            for b in range(_K):
                pltpu.make_async_copy(
                    table_hbm.at[idx_v.at[0]], rows[h * _K + b], gsems[h]
                ).wait()

        def start_writebacks(g, h):
            for b in range(_K):
                j = g * _K + b
                off = base + j * _CHUNK
                pltpu.async_copy(rows[h * _K + b], out_hbm.at[pl.ds(off, _CHUNK)], wsems[h])

        def drain_writebacks(h):
            for b in range(_K):
                pltpu.make_async_copy(
                    rows[h * _K + b], out_hbm.at[pl.ds(0, _CHUNK)], wsems[h]
                ).wait()

        def half_step(g, h, o):
            drain_gathers(h)

            @pl.when(g + 1 < n_groups)
            def _():
                @pl.when(g >= 1)
                def _():
                    drain_writebacks(o)

                start_gathers(g + 1, o)

            start_writebacks(g, h)

        start_gathers(0, 0)

        def body(g, carry):
            half_step(g, 0, 1)
            half_step(g + 1, 1, 0)
            return carry

        lax.fori_loop(0, n_groups // 2, lambda i, c: body(i * 2, c), 0)
        drain_writebacks(1)

    return gather_kernel


def _seq_lens_body(x_ref, o_ref):
    o_ref[...] = jnp.sum((x_ref[...] != 0).astype(jnp.int32), axis=1)


def kernel(x, table):
    B_, L_ = x.shape
    V, D = table.shape
    N = B_ * L_
    x_i32 = x.astype(jnp.int32)
    per_w = N // _NW
    idx3 = x_i32.reshape(_NW, per_w // _CHUNK, _CHUNK)

    emb_flat = _make_gather(V, D, N)(table, idx3)
    emb = emb_flat.reshape(B_, L_, D)

    seq_lens = pl.pallas_call(
        _seq_lens_body,
        out_shape=jax.ShapeDtypeStruct((B_,), jnp.int32),
    )(x_i32)

    return (emb, seq_lens)
